# baseline (speedup 1.0000x reference)
import functools
import math

import jax
import jax.numpy as jnp
from jax import lax
from jax.experimental import pallas as pl
from jax.experimental.pallas import tpu as pltpu
from jax.experimental.pallas import tpu_sc as plsc

NC = 2
NS = 16
NW = NC * NS
CHUNK = 512
PSTRIDE = 33


def _proj_body(x_ref, wq_ref, wk_ref, wv_ref, q_ref, k_ref, v_ref, *, inv_dk):
    x = x_ref[...]

    def dot(w):
        return lax.dot_general(x, w, (((1,), (0,)), ((), ())),
                               precision=lax.Precision.HIGHEST,
                               preferred_element_type=jnp.float32)

    q_ref[...] = dot(wq_ref[...]) * inv_dk
    k_ref[...] = dot(wk_ref[...])
    v_ref[...] = dot(wv_ref[...])


def _edge_body(q_hbm, k_hbm, v_hbm, row_hbm, col_hbm, zn_hbm, zd_hbm,
               num_hbm, den_hbm,
               rowi, coli, qb, kb, vb, pbuf, wb,
               num_sp, den_sp, sem,
               *, e_total, cpw, rpt, nheads):
    c = lax.axis_index("c")
    s = lax.axis_index("s")
    wid = c * NS + s

    r0 = s * rpt
    pltpu.sync_copy(zn_hbm, num_sp.at[pl.ds(r0, rpt)])
    pltpu.sync_copy(zd_hbm, den_sp.at[pl.ds(r0, rpt)])
    pltpu.sync_copy(row_hbm.at[wid], rowi)
    pltpu.sync_copy(col_hbm.at[wid], coli)
    plsc.subcore_barrier()

    lane = lax.iota(jnp.int32, 16)
    ebase0 = wid * (cpw * CHUNK)

    def gather_start(j, qd, kd, vd):
        pltpu.make_async_copy(q_hbm.at[rowi.at[j]], qd, sem).start()
        pltpu.make_async_copy(k_hbm.at[coli.at[j]], kd, sem).start()
        pltpu.make_async_copy(v_hbm.at[coli.at[j]], vd, sem).start()

    def gather_wait(j, qd, kd, vd):
        pltpu.make_async_copy(q_hbm.at[rowi.at[j]], qd, sem).wait()
        pltpu.make_async_copy(k_hbm.at[coli.at[j]], kd, sem).wait()
        pltpu.make_async_copy(v_hbm.at[coli.at[j]], vd, sem).wait()

    nvec = CHUNK * nheads // 16

    def compute_scatter(j, qd, kd, vd):
        gbase = ebase0 + j * CHUNK

        def pstep(e, carry2):
            pbuf[pl.ds(e * PSTRIDE, 16)] = (qd[e, pl.ds(0, 16)]
                                            * kd[e, pl.ds(0, 16)])
            pbuf[pl.ds(e * PSTRIDE + 16, 16)] = (qd[e, pl.ds(16, 16)]
                                                 * kd[e, pl.ds(16, 16)])
            return carry2

        lax.fori_loop(0, CHUNK, pstep, 0, unroll=16)

        dnums = lax.GatherDimensionNumbers(
            offset_dims=(), collapsed_slice_dims=(0,), start_index_map=(0,))

        def gstep(g, carry2):
            fidx = (g * 16 + lane) * PSTRIDE
            accs = [jnp.zeros((16,), jnp.float32) for _ in range(4)]
            for h in range(nheads):
                accs[h % 4] = accs[h % 4] + plsc.load_gather(pbuf, [fidx + h])
            w = jnp.exp((accs[0] + accs[1]) + (accs[2] + accs[3]))
            ge = gbase + g * 16 + lane
            w = jnp.where(ge < e_total, w, 0.0)
            wb[pl.ds(g * 16, 16)] = w
            for ee in range(16):
                we = lax.gather(w, jnp.full((16, 1), ee, jnp.int32), dnums,
                                slice_sizes=(1,),
                                mode=lax.GatherScatterMode.PROMISE_IN_BOUNDS)
                e = g * 16 + ee
                vd[e, pl.ds(0, 16)] = vd[e, pl.ds(0, 16)] * we
                vd[e, pl.ds(16, 16)] = vd[e, pl.ds(16, 16)] * we
            return carry2

        lax.fori_loop(0, CHUNK // 16, gstep, 0, unroll=2)
        pltpu.sync_copy(vd, num_sp.at[rowi.at[j]], add=True)
        pltpu.sync_copy(wb, den_sp.at[rowi.at[j]], add=True)

    def chunk(j, carry):
        gather_start(j, qb, kb, vb)
        gather_wait(j, qb, kb, vb)
        compute_scatter(j, qb, kb, vb)
        return carry

    lax.fori_loop(0, cpw, chunk, 0)
    plsc.subcore_barrier()
    n_pad = den_sp.shape[0]
    pltpu.sync_copy(num_sp.at[pl.ds(r0, rpt)], num_hbm.at[c, pl.ds(r0, rpt)])
    pltpu.sync_copy(den_sp.at[pl.ds(r0, rpt)],
                    den_hbm.at[pl.ds(c * n_pad + r0, rpt)])


def _combine_body(num_ref, den_ref, out_ref):
    nrows = out_ref.shape[0]
    n = num_ref[...]
    d = den_ref[...]
    ns = (n[0] + n[1])[:nrows]
    ds = (d[0] + d[1])[:nrows]
    ok = ds > 0.0
    safe = jnp.where(ok, ds, 1.0)
    out_ref[...] = jnp.where(ok, ns / safe, 0.0)


def kernel(X, edge_index, Wq, Wk, Wv):
    n, d = X.shape
    h = Wq.shape[1]
    e = edge_index.shape[1]
    inv_dk = 1.0 / math.sqrt(float(h))

    bn = 1000
    grid = (n // bn,)
    q, k, v = pl.pallas_call(
        functools.partial(_proj_body, inv_dk=inv_dk),
        grid=grid,
        in_specs=[
            pl.BlockSpec((bn, d), lambda i: (i, 0)),
            pl.BlockSpec((d, h), lambda i: (0, 0)),
            pl.BlockSpec((d, h), lambda i: (0, 0)),
            pl.BlockSpec((d, h), lambda i: (0, 0)),
        ],
        out_specs=[
            pl.BlockSpec((bn, h), lambda i: (i, 0)),
            pl.BlockSpec((bn, h), lambda i: (i, 0)),
            pl.BlockSpec((bn, h), lambda i: (i, 0)),
        ],
        out_shape=[
            jax.ShapeDtypeStruct((n, h), jnp.float32),
            jax.ShapeDtypeStruct((n, h), jnp.float32),
            jax.ShapeDtypeStruct((n, h), jnp.float32),
        ],
    )(X, Wq, Wk, Wv)

    cpw = -(-e // (NW * CHUNK))
    e_pad = NW * cpw * CHUNK
    rpt = -(-(-(-n // NS)) // 8) * 8
    n_pad = NS * rpt
    row = jnp.pad(edge_index[0], (0, e_pad - e)).reshape(NW, cpw, CHUNK)
    col = jnp.pad(edge_index[1], (0, e_pad - e)).reshape(NW, cpw, CHUNK)
    zn = jnp.zeros((rpt, h), jnp.float32)
    zd = jnp.zeros((rpt,), jnp.float32)

    mesh = plsc.VectorSubcoreMesh(core_axis_name="c", subcore_axis_name="s",
                                  num_cores=NC, num_subcores=NS)
    edge_fn = pl.kernel(
        functools.partial(_edge_body, e_total=e, cpw=cpw, rpt=rpt, nheads=h),
        out_type=(
            jax.ShapeDtypeStruct((NC, n_pad, h), jnp.float32),
            jax.ShapeDtypeStruct((NC * n_pad,), jnp.float32),
        ),
        mesh=mesh,
        compiler_params=pltpu.CompilerParams(needs_layout_passes=False,
                                             use_tc_tiling_on_sc=False),
        scratch_types=[
            pltpu.VMEM((cpw, CHUNK), jnp.int32),
            pltpu.VMEM((cpw, CHUNK), jnp.int32),
            pltpu.VMEM((CHUNK, h), jnp.float32),
            pltpu.VMEM((CHUNK, h), jnp.float32),
            pltpu.VMEM((CHUNK, h), jnp.float32),
            pltpu.VMEM((CHUNK * PSTRIDE + 16,), jnp.float32),
            pltpu.VMEM((CHUNK,), jnp.float32),
            pltpu.VMEM_SHARED((n_pad, h), jnp.float32),
            pltpu.VMEM_SHARED((n_pad,), jnp.float32),
            pltpu.SemaphoreType.DMA,
        ],
    )
    num, den = edge_fn(q, k, v, row, col, zn, zd)

    out = pl.pallas_call(
        _combine_body,
        out_shape=jax.ShapeDtypeStruct((n, h), jnp.float32),
    )(num, den.reshape(NC, n_pad, 1))
    return out


# speedup vs baseline: 1.0300x; 1.0300x over previous
import functools
import math

import jax
import jax.numpy as jnp
from jax import lax
from jax.experimental import pallas as pl
from jax.experimental.pallas import tpu as pltpu
from jax.experimental.pallas import tpu_sc as plsc

NC = 2
NS = 16
NW = NC * NS
CHUNK = 512
PSTRIDE = 33


def _proj_body(x_ref, wq_ref, wkv_ref, q_ref, kv_ref, *, inv_dk):
    x = x_ref[...]

    def dot(w):
        return lax.dot_general(x, w, (((1,), (0,)), ((), ())),
                               preferred_element_type=jnp.float32)

    q_ref[...] = dot(wq_ref[...]) * inv_dk
    kv_ref[...] = dot(wkv_ref[...])


def _edge_body(q_hbm, kv_hbm, row_hbm, col_hbm, zn_hbm, zd_hbm,
               num_hbm, den_hbm,
               rowi, coli, qb, kvb, vb, pbuf, wb,
               num_sp, den_sp, sem,
               *, e_total, cpw, rpt, nheads):
    c = lax.axis_index("c")
    s = lax.axis_index("s")
    wid = c * NS + s

    r0 = s * rpt
    pltpu.sync_copy(zn_hbm, num_sp.at[pl.ds(r0, rpt)])
    pltpu.sync_copy(zd_hbm, den_sp.at[pl.ds(r0, rpt)])
    ibase = wid * cpw * CHUNK
    for j in range(cpw):
        pltpu.make_async_copy(row_hbm.at[pl.ds(ibase + j * CHUNK, CHUNK)],
                              rowi.at[j], sem).start()
        pltpu.make_async_copy(col_hbm.at[pl.ds(ibase + j * CHUNK, CHUNK)],
                              coli.at[j], sem).start()
    for j in range(cpw):
        pltpu.make_async_copy(row_hbm.at[pl.ds(ibase + j * CHUNK, CHUNK)],
                              rowi.at[j], sem).wait()
        pltpu.make_async_copy(col_hbm.at[pl.ds(ibase + j * CHUNK, CHUNK)],
                              coli.at[j], sem).wait()
    plsc.subcore_barrier()

    lane = lax.iota(jnp.int32, 16)
    ebase0 = wid * (cpw * CHUNK)

    def gather_start(j):
        pltpu.make_async_copy(q_hbm.at[rowi.at[j]], qb, sem).start()
        pltpu.make_async_copy(kv_hbm.at[coli.at[j]], kvb, sem).start()

    def gather_wait(j):
        pltpu.make_async_copy(q_hbm.at[rowi.at[j]], qb, sem).wait()
        pltpu.make_async_copy(kv_hbm.at[coli.at[j]], kvb, sem).wait()

    def compute_scatter(j):
        gbase = ebase0 + j * CHUNK

        def pstep(e, carry2):
            pbuf[pl.ds(e * PSTRIDE, 16)] = (qb[e, pl.ds(0, 16)]
                                            * kvb[e, pl.ds(0, 16)])
            pbuf[pl.ds(e * PSTRIDE + 16, 16)] = (qb[e, pl.ds(16, 16)]
                                                 * kvb[e, pl.ds(16, 16)])
            return carry2

        lax.fori_loop(0, CHUNK, pstep, 0, unroll=16)

        def gstep(g, carry2):
            fidx = (g * 16 + lane) * PSTRIDE
            accs = [jnp.zeros((16,), jnp.float32) for _ in range(4)]
            for h in range(nheads):
                accs[h % 4] = accs[h % 4] + plsc.load_gather(pbuf, [fidx + h])
            w = jnp.exp((accs[0] + accs[1]) + (accs[2] + accs[3]))
            ge = gbase + g * 16 + lane
            w = jnp.where(ge < e_total, w, 0.0)
            wb[pl.ds(g * 16, 16)] = w
            return carry2

        lax.fori_loop(0, CHUNK // 16, gstep, 0, unroll=2)

        def estep(e, carry2):
            we = plsc.load_gather(wb, [jnp.full((16,), e, jnp.int32)])
            vb[e, pl.ds(0, 16)] = kvb[e, pl.ds(nheads, 16)] * we
            vb[e, pl.ds(16, 16)] = kvb[e, pl.ds(nheads + 16, 16)] * we
            return carry2

        lax.fori_loop(0, CHUNK, estep, 0, unroll=16)
        pltpu.sync_copy(vb, num_sp.at[rowi.at[j]], add=True)
        pltpu.sync_copy(wb, den_sp.at[rowi.at[j]], add=True)

    def chunk(j, carry):
        gather_start(j)
        gather_wait(j)
        compute_scatter(j)
        return carry

    lax.fori_loop(0, cpw, chunk, 0)
    plsc.subcore_barrier()
    n_pad = den_sp.shape[0]
    pltpu.sync_copy(num_sp.at[pl.ds(r0, rpt)], num_hbm.at[c, pl.ds(r0, rpt)])
    pltpu.sync_copy(den_sp.at[pl.ds(r0, rpt)],
                    den_hbm.at[pl.ds(c * n_pad + r0, rpt)])


def _combine_body(num_ref, den_ref, out_ref):
    n = num_ref[...]
    d = den_ref[...]
    ns = n[0] + n[1]
    ds = d[0] + d[1]
    ok = ds > 0.0
    safe = jnp.where(ok, ds, 1.0)
    out_ref[...] = jnp.where(ok, ns / safe, 0.0)


def kernel(X, edge_index, Wq, Wk, Wv):
    n, d = X.shape
    h = Wq.shape[1]
    e = edge_index.shape[1]
    inv_dk = 1.0 / math.sqrt(float(h))

    bn = 1000
    grid = (n // bn,)
    wkv = jnp.concatenate([Wk, Wv], axis=1)
    q, kv = pl.pallas_call(
        functools.partial(_proj_body, inv_dk=inv_dk),
        grid=grid,
        in_specs=[
            pl.BlockSpec((bn, d), lambda i: (i, 0)),
            pl.BlockSpec((d, h), lambda i: (0, 0)),
            pl.BlockSpec((d, 2 * h), lambda i: (0, 0)),
        ],
        out_specs=[
            pl.BlockSpec((bn, h), lambda i: (i, 0)),
            pl.BlockSpec((bn, 2 * h), lambda i: (i, 0)),
        ],
        out_shape=[
            jax.ShapeDtypeStruct((n, h), jnp.float32),
            jax.ShapeDtypeStruct((n, 2 * h), jnp.float32),
        ],
    )(X, Wq, wkv)

    cpw = -(-e // (NW * CHUNK))
    e_pad = NW * cpw * CHUNK
    rpt = -(-(-(-n // NS)) // 8) * 8
    n_pad = NS * rpt
    row = jnp.pad(edge_index[0], (0, e_pad - e))
    col = jnp.pad(edge_index[1], (0, e_pad - e))
    zn = jnp.zeros((rpt, h), jnp.float32)
    zd = jnp.zeros((rpt,), jnp.float32)

    mesh = plsc.VectorSubcoreMesh(core_axis_name="c", subcore_axis_name="s",
                                  num_cores=NC, num_subcores=NS)
    edge_fn = pl.kernel(
        functools.partial(_edge_body, e_total=e, cpw=cpw, rpt=rpt, nheads=h),
        out_type=(
            jax.ShapeDtypeStruct((NC, n_pad, h), jnp.float32),
            jax.ShapeDtypeStruct((NC * n_pad,), jnp.float32),
        ),
        mesh=mesh,
        compiler_params=pltpu.CompilerParams(needs_layout_passes=False,
                                             use_tc_tiling_on_sc=False),
        scratch_types=[
            pltpu.VMEM((cpw, CHUNK), jnp.int32),
            pltpu.VMEM((cpw, CHUNK), jnp.int32),
            pltpu.VMEM((CHUNK, h), jnp.float32),
            pltpu.VMEM((CHUNK, 2 * h), jnp.float32),
            pltpu.VMEM((CHUNK, h), jnp.float32),
            pltpu.VMEM((CHUNK * PSTRIDE + 16,), jnp.float32),
            pltpu.VMEM((CHUNK,), jnp.float32),
            pltpu.VMEM_SHARED((n_pad, h), jnp.float32),
            pltpu.VMEM_SHARED((n_pad,), jnp.float32),
            pltpu.SemaphoreType.DMA,
        ],
    )
    num, den = edge_fn(q, kv, row, col, zn, zd)

    out = pl.pallas_call(
        _combine_body,
        grid=(n // bn,),
        in_specs=[
            pl.BlockSpec((NC, bn, h), lambda i: (0, i, 0)),
            pl.BlockSpec((NC, bn, 1), lambda i: (0, i, 0)),
        ],
        out_specs=pl.BlockSpec((bn, h), lambda i: (i, 0)),
        out_shape=jax.ShapeDtypeStruct((n, h), jnp.float32),
    )(num, den.reshape(NC, n_pad, 1))
    return out


# speedup vs baseline: 1.3302x; 1.2914x over previous
import functools
import math

import jax
import jax.numpy as jnp
from jax import lax
from jax.experimental import pallas as pl
from jax.experimental.pallas import tpu as pltpu
from jax.experimental.pallas import tpu_sc as plsc

NC = 2
NS = 16
NW = NC * NS
CHUNK = 512
PSTRIDE = 33


def _proj_body(x_ref, wq_ref, wkv_ref, q_ref, kv_ref, *, inv_dk):
    x = x_ref[...]

    def dot(w):
        return lax.dot_general(x, w, (((1,), (0,)), ((), ())),
                               preferred_element_type=jnp.float32)

    q_ref[...] = dot(wq_ref[...]) * inv_dk
    kv_ref[...] = dot(wkv_ref[...])


def _edge_body(q_hbm, kv_hbm, row_hbm, col_hbm, zn_hbm, zd_hbm,
               num_hbm, den_hbm,
               rowi, coli, qb, kvb, vb, pbuf, wb,
               num_sp, den_sp, sem,
               *, e_total, cpw, rpt, nheads):
    c = lax.axis_index("c")
    s = lax.axis_index("s")
    wid = c * NS + s

    r0 = s * rpt
    pltpu.sync_copy(zn_hbm, num_sp.at[pl.ds(r0, rpt)])
    pltpu.sync_copy(zd_hbm, den_sp.at[pl.ds(r0, rpt)])
    ibase = wid * cpw * CHUNK
    for j in range(cpw):
        pltpu.make_async_copy(row_hbm.at[pl.ds(ibase + j * CHUNK, CHUNK)],
                              rowi.at[j], sem).start()
        pltpu.make_async_copy(col_hbm.at[pl.ds(ibase + j * CHUNK, CHUNK)],
                              coli.at[j], sem).start()
    for j in range(cpw):
        pltpu.make_async_copy(row_hbm.at[pl.ds(ibase + j * CHUNK, CHUNK)],
                              rowi.at[j], sem).wait()
        pltpu.make_async_copy(col_hbm.at[pl.ds(ibase + j * CHUNK, CHUNK)],
                              coli.at[j], sem).wait()
    plsc.subcore_barrier()

    lane = lax.iota(jnp.int32, 16)
    ebase0 = wid * (cpw * CHUNK)

    def gather_start(j):
        pltpu.make_async_copy(q_hbm.at[rowi.at[j]], qb, sem).start()
        pltpu.make_async_copy(kv_hbm.at[coli.at[j]], kvb, sem).start()

    def gather_wait(j):
        pltpu.make_async_copy(q_hbm.at[rowi.at[j]], qb, sem).wait()
        pltpu.make_async_copy(kv_hbm.at[coli.at[j]], kvb, sem).wait()

    def compute_scatter(j):
        gbase = ebase0 + j * CHUNK

        def pstep(e, carry2):
            pbuf[pl.ds(e * PSTRIDE, 16)] = (qb[e, pl.ds(0, 16)]
                                            * kvb[e, pl.ds(0, 16)])
            pbuf[pl.ds(e * PSTRIDE + 16, 16)] = (qb[e, pl.ds(16, 16)]
                                                 * kvb[e, pl.ds(16, 16)])
            return carry2

        lax.fori_loop(0, CHUNK, pstep, 0, unroll=16)

        def gstep(g, carry2):
            fidx = (g * 16 + lane) * PSTRIDE
            accs = [jnp.zeros((16,), jnp.float32) for _ in range(4)]
            for h in range(nheads):
                accs[h % 4] = accs[h % 4] + plsc.load_gather(pbuf, [fidx + h])
            w = jnp.exp((accs[0] + accs[1]) + (accs[2] + accs[3]))
            ge = gbase + g * 16 + lane
            w = jnp.where(ge < e_total, w, 0.0)
            wb[pl.ds(g * 16, 16)] = w
            return carry2

        lax.fori_loop(0, CHUNK // 16, gstep, 0, unroll=2)

        def estep(e, carry2):
            we = plsc.load_gather(wb, [jnp.full((16,), e, jnp.int32)])
            vb[e, pl.ds(0, 16)] = kvb[e, pl.ds(nheads, 16)] * we
            vb[e, pl.ds(16, 16)] = kvb[e, pl.ds(nheads + 16, 16)] * we
            return carry2

        lax.fori_loop(0, CHUNK, estep, 0, unroll=16)
        pltpu.sync_copy(vb, num_sp.at[rowi.at[j]], add=True)
        pltpu.sync_copy(wb, den_sp.at[rowi.at[j]], add=True)

    def chunk(j, carry):
        gather_start(j)
        gather_wait(j)
        compute_scatter(j)
        return carry

    lax.fori_loop(0, cpw, chunk, 0)
    plsc.subcore_barrier()
    n_pad = den_sp.shape[0]
    pltpu.sync_copy(num_sp.at[pl.ds(r0, rpt)], num_hbm.at[c, pl.ds(r0, rpt)])
    pltpu.sync_copy(den_sp.at[pl.ds(r0, rpt)],
                    den_hbm.at[pl.ds(c * n_pad + r0, rpt)])


def _combine_body(num_ref, den_ref, out_ref):
    n = num_ref[...]
    d = den_ref[...]
    ns = n[0] + n[1]
    ds = d[0] + d[1]
    ok = ds > 0.0
    safe = jnp.where(ok, ds, 1.0)
    out_ref[...] = jnp.where(ok, ns / safe, 0.0)


def kernel(X, edge_index, Wq, Wk, Wv):
    n, d = X.shape
    h = Wq.shape[1]
    e = edge_index.shape[1]
    inv_dk = 1.0 / math.sqrt(float(h))

    bn = 1000
    grid = (n // bn,)
    wkv = jnp.concatenate([Wk, Wv], axis=1)
    q, kv = pl.pallas_call(
        functools.partial(_proj_body, inv_dk=inv_dk),
        grid=grid,
        in_specs=[
            pl.BlockSpec((bn, d), lambda i: (i, 0)),
            pl.BlockSpec((d, h), lambda i: (0, 0)),
            pl.BlockSpec((d, 2 * h), lambda i: (0, 0)),
        ],
        out_specs=[
            pl.BlockSpec((bn, h), lambda i: (i, 0)),
            pl.BlockSpec((bn, 2 * h), lambda i: (i, 0)),
        ],
        out_shape=[
            jax.ShapeDtypeStruct((n, h), jnp.float32),
            jax.ShapeDtypeStruct((n, 2 * h), jnp.float32),
        ],
    )(X, Wq, wkv)

    cpw = -(-e // (NW * CHUNK))
    e_pad = NW * cpw * CHUNK
    rpt = -(-(-(-n // NS)) // 8) * 8
    n_pad = NS * rpt
    spread = (jnp.arange(e_pad - e, dtype=jnp.int32) * 8) % n
    row = jnp.concatenate([edge_index[0], spread])
    col = jnp.concatenate([edge_index[1], spread])
    zn = jnp.zeros((rpt, h), jnp.float32)
    zd = jnp.zeros((rpt,), jnp.float32)

    mesh = plsc.VectorSubcoreMesh(core_axis_name="c", subcore_axis_name="s",
                                  num_cores=NC, num_subcores=NS)
    edge_fn = pl.kernel(
        functools.partial(_edge_body, e_total=e, cpw=cpw, rpt=rpt, nheads=h),
        out_type=(
            jax.ShapeDtypeStruct((NC, n_pad, h), jnp.float32),
            jax.ShapeDtypeStruct((NC * n_pad,), jnp.float32),
        ),
        mesh=mesh,
        compiler_params=pltpu.CompilerParams(needs_layout_passes=False,
                                             use_tc_tiling_on_sc=False),
        scratch_types=[
            pltpu.VMEM((cpw, CHUNK), jnp.int32),
            pltpu.VMEM((cpw, CHUNK), jnp.int32),
            pltpu.VMEM((CHUNK, h), jnp.float32),
            pltpu.VMEM((CHUNK, 2 * h), jnp.float32),
            pltpu.VMEM((CHUNK, h), jnp.float32),
            pltpu.VMEM((CHUNK * PSTRIDE + 16,), jnp.float32),
            pltpu.VMEM((CHUNK,), jnp.float32),
            pltpu.VMEM_SHARED((n_pad, h), jnp.float32),
            pltpu.VMEM_SHARED((n_pad,), jnp.float32),
            pltpu.SemaphoreType.DMA,
        ],
    )
    num, den = edge_fn(q, kv, row, col, zn, zd)

    out = pl.pallas_call(
        _combine_body,
        grid=(n // bn,),
        in_specs=[
            pl.BlockSpec((NC, bn, h), lambda i: (0, i, 0)),
            pl.BlockSpec((NC, bn, 1), lambda i: (0, i, 0)),
        ],
        out_specs=pl.BlockSpec((bn, h), lambda i: (i, 0)),
        out_shape=jax.ShapeDtypeStruct((n, h), jnp.float32),
    )(num, den.reshape(NC, n_pad, 1))
    return out


# speedup vs baseline: 1.3653x; 1.0264x over previous
import functools
import math

import jax
import jax.numpy as jnp
from jax import lax
from jax.experimental import pallas as pl
from jax.experimental.pallas import tpu as pltpu
from jax.experimental.pallas import tpu_sc as plsc

NC = 2
NS = 16
NW = NC * NS
CHUNK = 512
PSTRIDE = 33


def _proj_body(x_ref, wq_ref, wkv_ref, q_ref, kv_ref, *, inv_dk):
    x = x_ref[...]

    def dot(w):
        return lax.dot_general(x, w, (((1,), (0,)), ((), ())),
                               preferred_element_type=jnp.float32)

    q_ref[...] = dot(wq_ref[...]) * inv_dk
    kv_ref[...] = dot(wkv_ref[...])


def _edge_body(q_hbm, kv_hbm, row_hbm, col_hbm, zn_hbm, zd_hbm,
               num_hbm, den_hbm,
               rowi, coli, qb, kvb, vb, pbuf, wb, dbuf,
               num_sp, den_sp, sem,
               *, e_total, cpw, rpt, nheads):
    c = lax.axis_index("c")
    s = lax.axis_index("s")
    wid = c * NS + s

    r0 = s * rpt
    pltpu.sync_copy(zn_hbm, num_sp.at[pl.ds(r0, rpt)])
    pltpu.sync_copy(zd_hbm, den_sp.at[pl.ds(r0, rpt)])
    ibase = wid * cpw * CHUNK
    for j in range(cpw):
        pltpu.make_async_copy(row_hbm.at[pl.ds(ibase + j * CHUNK, CHUNK)],
                              rowi.at[j], sem).start()
        pltpu.make_async_copy(col_hbm.at[pl.ds(ibase + j * CHUNK, CHUNK)],
                              coli.at[j], sem).start()
    for j in range(cpw):
        pltpu.make_async_copy(row_hbm.at[pl.ds(ibase + j * CHUNK, CHUNK)],
                              rowi.at[j], sem).wait()
        pltpu.make_async_copy(col_hbm.at[pl.ds(ibase + j * CHUNK, CHUNK)],
                              coli.at[j], sem).wait()
    plsc.subcore_barrier()

    lane = lax.iota(jnp.int32, 16)
    ebase0 = wid * (cpw * CHUNK)

    def gather_start(j):
        pltpu.make_async_copy(q_hbm.at[rowi.at[j]], qb, sem).start()
        pltpu.make_async_copy(kv_hbm.at[coli.at[j]], kvb, sem).start()

    def gather_wait(j):
        pltpu.make_async_copy(q_hbm.at[rowi.at[j]], qb, sem).wait()
        pltpu.make_async_copy(kv_hbm.at[coli.at[j]], kvb, sem).wait()

    def compute_scatter(j):
        gbase = ebase0 + j * CHUNK

        def pstep(e, carry2):
            pbuf[pl.ds(e * PSTRIDE, 16)] = (qb[e, pl.ds(0, 16)]
                                            * kvb[e, pl.ds(0, 16)])
            pbuf[pl.ds(e * PSTRIDE + 16, 16)] = (qb[e, pl.ds(16, 16)]
                                                 * kvb[e, pl.ds(16, 16)])
            return carry2

        lax.fori_loop(0, CHUNK, pstep, 0, unroll=16)

        def gstep(g, carry2):
            fidx = (g * 16 + lane) * PSTRIDE
            accs = [jnp.zeros((16,), jnp.float32) for _ in range(4)]
            for h in range(nheads):
                accs[h % 4] = accs[h % 4] + plsc.load_gather(pbuf, [fidx + h])
            w = jnp.exp((accs[0] + accs[1]) + (accs[2] + accs[3]))
            ge = gbase + g * 16 + lane
            w = jnp.where(ge < e_total, w, 0.0)
            wb[pl.ds(g * 16, 16)] = w
            return carry2

        lax.fori_loop(0, CHUNK // 16, gstep, 0, unroll=2)

        def estep(e, carry2):
            we = plsc.load_gather(wb, [jnp.full((16,), e, jnp.int32)])
            vb[e, pl.ds(0, 16)] = kvb[e, pl.ds(nheads, 16)] * we
            vb[e, pl.ds(16, 16)] = kvb[e, pl.ds(nheads + 16, 16)] * we
            return carry2

        lax.fori_loop(0, CHUNK, estep, 0, unroll=16)
        pltpu.sync_copy(vb, num_sp.at[rowi.at[j]], add=True)
        pltpu.sync_copy(wb, den_sp.at[rowi.at[j]], add=True)

    def chunk(j, carry):
        gather_start(j)
        gather_wait(j)
        compute_scatter(j)
        return carry

    lax.fori_loop(0, cpw, chunk, 0)
    plsc.subcore_barrier()
    pltpu.sync_copy(num_sp.at[pl.ds(r0, rpt)], num_hbm.at[c, pl.ds(r0, rpt)])
    pltpu.sync_copy(den_sp.at[pl.ds(r0, rpt)], dbuf.at[pl.ds(0, rpt)])
    half0 = rpt - (rpt // 2 // 8) * 8
    for hbase, hlen in ((0, half0), (half0, rpt - half0)):
        def xstep(r, carry2):
            dv = plsc.load_gather(dbuf, [jnp.full((16,), hbase, jnp.int32) + r])
            vb[r, pl.ds(0, 16)] = dv
            vb[r, pl.ds(16, 16)] = dv
            return carry2

        lax.fori_loop(0, hlen, xstep, 0, unroll=8)
        pltpu.sync_copy(vb.at[pl.ds(0, hlen)],
                        den_hbm.at[c, pl.ds(r0 + hbase, hlen)])


def _combine_body(num_ref, den_ref, out_ref):
    n = num_ref[...]
    d = den_ref[...]
    ns = n[0] + n[1]
    ds = d[0] + d[1]
    ok = ds > 0.0
    safe = jnp.where(ok, ds, 1.0)
    out_ref[...] = jnp.where(ok, ns / safe, 0.0)


def kernel(X, edge_index, Wq, Wk, Wv):
    n, d = X.shape
    h = Wq.shape[1]
    e = edge_index.shape[1]
    inv_dk = 1.0 / math.sqrt(float(h))

    bn = 1000
    grid = (n // bn,)
    wkv = jnp.concatenate([Wk, Wv], axis=1)
    q, kv = pl.pallas_call(
        functools.partial(_proj_body, inv_dk=inv_dk),
        grid=grid,
        in_specs=[
            pl.BlockSpec((bn, d), lambda i: (i, 0)),
            pl.BlockSpec((d, h), lambda i: (0, 0)),
            pl.BlockSpec((d, 2 * h), lambda i: (0, 0)),
        ],
        out_specs=[
            pl.BlockSpec((bn, h), lambda i: (i, 0)),
            pl.BlockSpec((bn, 2 * h), lambda i: (i, 0)),
        ],
        out_shape=[
            jax.ShapeDtypeStruct((n, h), jnp.float32),
            jax.ShapeDtypeStruct((n, 2 * h), jnp.float32),
        ],
    )(X, Wq, wkv)

    cpw = -(-e // (NW * CHUNK))
    e_pad = NW * cpw * CHUNK
    rpt = -(-(-(-n // NS)) // 8) * 8
    n_pad = NS * rpt
    spread = (jnp.arange(e_pad - e, dtype=jnp.int32) * 8) % n
    row = jnp.concatenate([edge_index[0], spread])
    col = jnp.concatenate([edge_index[1], spread])
    zn = jnp.zeros((rpt, h), jnp.float32)
    zd = jnp.zeros((rpt,), jnp.float32)

    mesh = plsc.VectorSubcoreMesh(core_axis_name="c", subcore_axis_name="s",
                                  num_cores=NC, num_subcores=NS)
    edge_fn = pl.kernel(
        functools.partial(_edge_body, e_total=e, cpw=cpw, rpt=rpt, nheads=h),
        out_type=(
            jax.ShapeDtypeStruct((NC, n_pad, h), jnp.float32),
            jax.ShapeDtypeStruct((NC, n_pad, h), jnp.float32),
        ),
        mesh=mesh,
        compiler_params=pltpu.CompilerParams(needs_layout_passes=False,
                                             use_tc_tiling_on_sc=False),
        scratch_types=[
            pltpu.VMEM((cpw, CHUNK), jnp.int32),
            pltpu.VMEM((cpw, CHUNK), jnp.int32),
            pltpu.VMEM((CHUNK, h), jnp.float32),
            pltpu.VMEM((CHUNK, 2 * h), jnp.float32),
            pltpu.VMEM((CHUNK, h), jnp.float32),
            pltpu.VMEM((CHUNK * PSTRIDE + 16,), jnp.float32),
            pltpu.VMEM((CHUNK,), jnp.float32),
            pltpu.VMEM((rpt + 8,), jnp.float32),
            pltpu.VMEM_SHARED((n_pad, h), jnp.float32),
            pltpu.VMEM_SHARED((n_pad,), jnp.float32),
            pltpu.SemaphoreType.DMA,
        ],
    )
    num, den = edge_fn(q, kv, row, col, zn, zd)

    out = pl.pallas_call(
        _combine_body,
        grid=(n // bn,),
        in_specs=[
            pl.BlockSpec((NC, bn, h), lambda i: (0, i, 0)),
            pl.BlockSpec((NC, bn, h), lambda i: (0, i, 0)),
        ],
        out_specs=pl.BlockSpec((bn, h), lambda i: (i, 0)),
        out_shape=jax.ShapeDtypeStruct((n, h), jnp.float32),
    )(num, den)
    return out
